# baseline (speedup 1.0000x reference)
import jax
import jax.numpy as jnp
import numpy as np
from jax import lax
from jax.experimental import pallas as pl
from jax.experimental.pallas import tpu as pltpu
from jax.experimental.pallas import tpu_sc as plsc

N = 10000
D = 256
E = 160000
HALF = 128
PACK = HALF // 2

NS = 16
CH = 32
NBUF = 4
PD = 3
E_PAD = 163840
EPT = E_PAD // NS
NSUP = 4
SCE = EPT // NSUP
CPS = SCE // CH
ACCR = 10240
RPT = ACCR // NS
WBC = 128

BLK = 2000

_HIMASK = np.uint32(0xFFFF0000)


def _segsum_half(x_h, out_h, c, s, scr, src_h, dst_h, w_h):
    src_all, dst_all, w_all = scr[0], scr[1], scr[2]
    srcv = scr[3:3 + NBUF]
    dstv = scr[3 + NBUF:3 + 2 * NBUF]
    prow = scr[3 + 2 * NBUF:3 + 3 * NBUF]
    urow = scr[3 + 3 * NBUF:3 + 4 * NBUF]
    acc = scr[3 + 4 * NBUF]
    gsem = scr[4 + 4 * NBUF:4 + 5 * NBUF]
    ssem = scr[4 + 5 * NBUF:4 + 6 * NBUF]
    base = s * EPT
    co = c * N
    oo = c * ACCR

    def prep(kk, b):
        for j in range(CH // 16):
            sl = pl.ds(j * 16, 16)
            esl = pl.ds(kk * CH + j * 16, 16)
            srcv[b][sl] = src_all[esl] + co
            dstv[b][sl] = dst_all[esl]

    def unpack_scale(b, kk):
        for g in range(CH // 16):
            wvec = w_all[pl.ds(kk * CH + g * 16, 16)]
            for i in range(16):
                r = g * 16 + i
                wi = wvec[i]
                for q in range(PACK // 16):
                    u = plsc.bitcast(prow[b][r, pl.ds(q * 16, 16)],
                                     jnp.uint32)
                    lo = plsc.bitcast(u << 16, jnp.float32)
                    hi = plsc.bitcast(u & _HIMASK, jnp.float32)
                    urow[b][r, pl.ds(q * 16, 16)] = lo * wi
                    urow[b][r, pl.ds(PACK + q * 16, 16)] = hi * wi

    @pl.loop(0, CH)
    def _zero_rows(i):
        for j in range(HALF // 16):
            urow[0][i, pl.ds(j * 16, 16)] = jnp.zeros((16,), jnp.float32)

    @pl.loop(0, RPT // CH)
    def _zero_acc(k):
        pltpu.sync_copy(urow[0], acc.at[pl.ds(s * RPT + k * CH, CH)])

    plsc.subcore_barrier()

    @pl.loop(0, NSUP)
    def _sup(m):
        moff = base + m * SCE
        pltpu.sync_copy(src_h.at[pl.ds(moff, SCE)], src_all)
        pltpu.sync_copy(dst_h.at[pl.ds(moff, SCE)], dst_all)
        pltpu.sync_copy(w_h.at[pl.ds(moff, SCE)], w_all)

        for d in range(PD):
            prep(d, d)
            pltpu.async_copy(x_h.at[srcv[d]], prow[d], gsem[d])

        @pl.loop(0, CPS, step=NBUF)
        def _chunk(k):
            for j in range(NBUF):
                cc = k + j
                jj = (j + PD) % NBUF

                @pl.when(cc + PD < CPS)
                def _():
                    @pl.when(cc + PD - NBUF >= 0)
                    def _():
                        pltpu.make_async_copy(
                            urow[jj], acc.at[dstv[jj]], ssem[jj]).wait()
                    prep(cc + PD, jj)
                    pltpu.async_copy(x_h.at[srcv[jj]], prow[jj], gsem[jj])

                pltpu.make_async_copy(x_h.at[srcv[j]], prow[j],
                                      gsem[j]).wait()
                unpack_scale(j, cc)
                pltpu.async_copy(urow[j], acc.at[dstv[j]], ssem[j],
                                 add=True)

        for j in range(NBUF):
            pltpu.make_async_copy(urow[j], acc.at[dstv[j]], ssem[j]).wait()

    plsc.subcore_barrier()

    @pl.loop(0, RPT // WBC)
    def _writeback(k):
        r0 = s * RPT + k * WBC
        pltpu.sync_copy(acc.at[pl.ds(r0, WBC)],
                        out_h.at[pl.ds(oo + r0, WBC)])


def _segsum_body(x2, src_h, dst_h, w_h, out2, *scr):
    c = lax.axis_index("c")
    s = lax.axis_index("s")
    _segsum_half(x2, out2, c, s, scr, src_h, dst_h, w_h)


_segsum = pl.kernel(
    _segsum_body,
    out_type=jax.ShapeDtypeStruct((2 * ACCR, HALF), jnp.float32),
    mesh=plsc.VectorSubcoreMesh(core_axis_name="c", subcore_axis_name="s"),
    compiler_params=pltpu.CompilerParams(use_tc_tiling_on_sc=False, needs_layout_passes=False),
    scratch_types=(
        [
            pltpu.VMEM((SCE,), jnp.int32),
            pltpu.VMEM((SCE,), jnp.int32),
            pltpu.VMEM((SCE,), jnp.float32),
        ]
        + [pltpu.VMEM((CH,), jnp.int32) for _ in range(NBUF)]
        + [pltpu.VMEM((CH,), jnp.int32) for _ in range(NBUF)]
        + [pltpu.VMEM((CH, PACK), jnp.float32) for _ in range(NBUF)]
        + [pltpu.VMEM((CH, HALF), jnp.float32) for _ in range(NBUF)]
        + [pltpu.VMEM_SHARED((ACCR, HALF), jnp.float32)]
        + [pltpu.SemaphoreType.DMA for _ in range(NBUF)]
        + [pltpu.SemaphoreType.DMA for _ in range(NBUF)]
    ),
)


def _sigmoid(x):
    return 1.0 / (1.0 + jnp.exp(-x))


def _pack_pairs(h):
    u = lax.bitcast_convert_type(h, jnp.uint32)
    packed = (u[:, :PACK] >> 16) | (u[:, PACK:] & _HIMASK)
    return lax.bitcast_convert_type(packed, jnp.float32)


def _stage1_body(alo, ahi, x, wr_lo, wr_hi, wx, b,
                 out_lo, out_hi, out_plo, out_phi):
    acc = jnp.dot(alo[...], wr_lo[...], preferred_element_type=jnp.float32)
    acc += jnp.dot(ahi[...], wr_hi[...], preferred_element_type=jnp.float32)
    acc += jnp.dot(x[...], wx[...], preferred_element_type=jnp.float32)
    ht = _sigmoid(acc + b[...])
    hlo = ht[:, :HALF]
    hhi = ht[:, HALF:]
    out_lo[...] = hlo
    out_hi[...] = hhi
    out_plo[...] = _pack_pairs(hlo)
    out_phi[...] = _pack_pairs(hhi)


_stage1 = pl.pallas_call(
    _stage1_body,
    grid=(N // BLK,),
    in_specs=[
        pl.BlockSpec((BLK, HALF), lambda i: (i, 0)),
        pl.BlockSpec((BLK, HALF), lambda i: (i, 0)),
        pl.BlockSpec((BLK, D), lambda i: (i, 0)),
        pl.BlockSpec((HALF, D), lambda i: (0, 0)),
        pl.BlockSpec((HALF, D), lambda i: (0, 0)),
        pl.BlockSpec((D, D), lambda i: (0, 0)),
        pl.BlockSpec((1, D), lambda i: (0, 0)),
    ],
    out_specs=[
        pl.BlockSpec((BLK, HALF), lambda i: (i, 0)),
        pl.BlockSpec((BLK, HALF), lambda i: (i, 0)),
        pl.BlockSpec((BLK, PACK), lambda i: (i, 0)),
        pl.BlockSpec((BLK, PACK), lambda i: (i, 0)),
    ],
    out_shape=[
        jax.ShapeDtypeStruct((N, HALF), jnp.float32),
        jax.ShapeDtypeStruct((N, HALF), jnp.float32),
        jax.ShapeDtypeStruct((N, PACK), jnp.float32),
        jax.ShapeDtypeStruct((N, PACK), jnp.float32),
    ],
)


def _stage2_body(alo, ahi, hlo, hhi, wr_lo, wr_hi, wx_lo, wx_hi, b, out):
    acc = jnp.dot(alo[...], wr_lo[...], preferred_element_type=jnp.float32)
    acc += jnp.dot(ahi[...], wr_hi[...], preferred_element_type=jnp.float32)
    acc += jnp.dot(hlo[...], wx_lo[...], preferred_element_type=jnp.float32)
    acc += jnp.dot(hhi[...], wx_hi[...], preferred_element_type=jnp.float32)
    out[...] = _sigmoid(acc + b[...])


_stage2 = pl.pallas_call(
    _stage2_body,
    grid=(N // BLK,),
    in_specs=[
        pl.BlockSpec((BLK, HALF), lambda i: (i, 0)),
        pl.BlockSpec((BLK, HALF), lambda i: (i, 0)),
        pl.BlockSpec((BLK, HALF), lambda i: (i, 0)),
        pl.BlockSpec((BLK, HALF), lambda i: (i, 0)),
        pl.BlockSpec((HALF, D), lambda i: (0, 0)),
        pl.BlockSpec((HALF, D), lambda i: (0, 0)),
        pl.BlockSpec((HALF, D), lambda i: (0, 0)),
        pl.BlockSpec((HALF, D), lambda i: (0, 0)),
        pl.BlockSpec((1, D), lambda i: (0, 0)),
    ],
    out_specs=pl.BlockSpec((BLK, D), lambda i: (i, 0)),
    out_shape=jax.ShapeDtypeStruct((N, D), jnp.float32),
)


@jax.jit
def kernel(X, edge_index, edge_weight,
           W_hx_rel, b_hx_rel, W_hx_root,
           W_hh_rel, b_hh_rel, W_hh_root,
           W_y_rel, b_y_rel, W_y_root):
    src = edge_index[0].astype(jnp.int32)
    dst = edge_index[1].astype(jnp.int32)
    w = edge_weight.astype(jnp.float32)

    pad = E_PAD - E
    src = jnp.concatenate([src, jnp.zeros((pad,), jnp.int32)])
    dst = jnp.concatenate([dst, jnp.zeros((pad,), jnp.int32)])
    w = jnp.concatenate([w, jnp.zeros((pad,), jnp.float32)])

    xp2 = jnp.concatenate([_pack_pairs(X[:, :HALF]),
                           _pack_pairs(X[:, HALF:])], axis=0)

    agg2 = _segsum(xp2, src, dst, w)
    agg_lo = agg2[:N]
    agg_hi = agg2[ACCR:ACCR + N]

    wr = W_hx_rel.T
    b1 = (b_hx_rel + b_hh_rel).reshape(1, D)
    ht_lo, ht_hi, ht_plo, ht_phi = _stage1(
        agg_lo, agg_hi, X, wr[:HALF], wr[HALF:], W_hx_root.T, b1)

    htp2 = jnp.concatenate([ht_plo, ht_phi], axis=0)
    ah2 = _segsum(htp2, src, dst, w)
    ah_lo = ah2[:N]
    ah_hi = ah2[ACCR:ACCR + N]

    wyr = W_y_rel.T
    wyx = W_y_root.T
    yt = _stage2(ah_lo, ah_hi, ht_lo, ht_hi,
                 wyr[:HALF], wyr[HALF:], wyx[:HALF], wyx[HALF:],
                 b_y_rel.reshape(1, D))
    return yt


# speedup vs baseline: 1.0967x; 1.0967x over previous
import jax
import jax.numpy as jnp
import numpy as np
from jax import lax
from jax.experimental import pallas as pl
from jax.experimental.pallas import tpu as pltpu
from jax.experimental.pallas import tpu_sc as plsc

N = 10000
D = 256
E = 160000
HALF = 128
PACK = HALF // 2

NS = 16
CH = 32
NBUF = 4
PD = 2
E_PAD = 163840
EPT = E_PAD // NS
NSUP = 2
SCE = EPT // NSUP
CPS = SCE // CH
ACCR = 10240
RPT = ACCR // NS
WBC = 128

BLK = 2000

_HIMASK = np.uint32(0xFFFF0000)


def _segsum_half(x_h, out_h, c, s, scr, src_h, dst_h, w_h):
    src_all, dst_all, w_all = scr[0], scr[1], scr[2]
    srcv = scr[3:3 + NBUF]
    dstv = scr[3 + NBUF:3 + 2 * NBUF]
    prow = scr[3 + 2 * NBUF:3 + 3 * NBUF]
    urow = scr[3 + 3 * NBUF:3 + 4 * NBUF]
    acc = scr[3 + 4 * NBUF]
    gsem = scr[4 + 4 * NBUF:4 + 5 * NBUF]
    ssem = scr[4 + 5 * NBUF:4 + 6 * NBUF]
    base = s * EPT
    co = c * N
    oo = c * ACCR

    def prep(kk, b):
        for j in range(CH // 16):
            sl = pl.ds(j * 16, 16)
            esl = pl.ds(kk * CH + j * 16, 16)
            srcv[b][sl] = src_all[esl] + co
            dstv[b][sl] = dst_all[esl]

    def unpack_scale(b, kk):
        for g in range(CH // 16):
            wvec = w_all[pl.ds(kk * CH + g * 16, 16)]
            for i in range(16):
                r = g * 16 + i
                wi = wvec[i]
                for q in range(PACK // 16):
                    u = plsc.bitcast(prow[b][r, pl.ds(q * 16, 16)],
                                     jnp.uint32)
                    lo = plsc.bitcast(u << 16, jnp.float32)
                    hi = plsc.bitcast(u & _HIMASK, jnp.float32)
                    urow[b][r, pl.ds(q * 16, 16)] = lo * wi
                    urow[b][r, pl.ds(PACK + q * 16, 16)] = hi * wi

    @pl.loop(0, CH)
    def _zero_rows(i):
        for j in range(HALF // 16):
            urow[0][i, pl.ds(j * 16, 16)] = jnp.zeros((16,), jnp.float32)

    @pl.loop(0, RPT // CH)
    def _zero_acc(k):
        pltpu.sync_copy(urow[0], acc.at[pl.ds(s * RPT + k * CH, CH)])

    plsc.subcore_barrier()

    @pl.loop(0, NSUP)
    def _sup(m):
        moff = base + m * SCE
        pltpu.sync_copy(src_h.at[pl.ds(moff, SCE)], src_all)
        pltpu.sync_copy(dst_h.at[pl.ds(moff, SCE)], dst_all)
        pltpu.sync_copy(w_h.at[pl.ds(moff, SCE)], w_all)

        for d in range(PD):
            prep(d, d)
            pltpu.async_copy(x_h.at[srcv[d]], prow[d], gsem[d])

        @pl.loop(0, CPS, step=NBUF)
        def _chunk(k):
            for j in range(NBUF):
                cc = k + j
                jj = (j + PD) % NBUF

                @pl.when(cc + PD < CPS)
                def _():
                    @pl.when(cc + PD - NBUF >= 0)
                    def _():
                        pltpu.make_async_copy(
                            urow[jj], acc.at[dstv[jj]], ssem[jj]).wait()
                    prep(cc + PD, jj)
                    pltpu.async_copy(x_h.at[srcv[jj]], prow[jj], gsem[jj])

                pltpu.make_async_copy(x_h.at[srcv[j]], prow[j],
                                      gsem[j]).wait()
                unpack_scale(j, cc)
                pltpu.async_copy(urow[j], acc.at[dstv[j]], ssem[j],
                                 add=True)

        for j in range(NBUF):
            pltpu.make_async_copy(urow[j], acc.at[dstv[j]], ssem[j]).wait()

    plsc.subcore_barrier()

    @pl.loop(0, RPT // WBC)
    def _writeback(k):
        r0 = s * RPT + k * WBC
        pltpu.sync_copy(acc.at[pl.ds(r0, WBC)],
                        out_h.at[pl.ds(oo + r0, WBC)])


def _segsum_body(x2, src_h, dst_h, w_h, out2, *scr):
    c = lax.axis_index("c")
    s = lax.axis_index("s")
    _segsum_half(x2, out2, c, s, scr, src_h, dst_h, w_h)


_segsum = pl.kernel(
    _segsum_body,
    out_type=jax.ShapeDtypeStruct((2 * ACCR, HALF), jnp.float32),
    mesh=plsc.VectorSubcoreMesh(core_axis_name="c", subcore_axis_name="s"),
    compiler_params=pltpu.CompilerParams(use_tc_tiling_on_sc=False, needs_layout_passes=False),
    scratch_types=(
        [
            pltpu.VMEM((SCE,), jnp.int32),
            pltpu.VMEM((SCE,), jnp.int32),
            pltpu.VMEM((SCE,), jnp.float32),
        ]
        + [pltpu.VMEM((CH,), jnp.int32) for _ in range(NBUF)]
        + [pltpu.VMEM((CH,), jnp.int32) for _ in range(NBUF)]
        + [pltpu.VMEM((CH, PACK), jnp.float32) for _ in range(NBUF)]
        + [pltpu.VMEM((CH, HALF), jnp.float32) for _ in range(NBUF)]
        + [pltpu.VMEM_SHARED((ACCR, HALF), jnp.float32)]
        + [pltpu.SemaphoreType.DMA for _ in range(NBUF)]
        + [pltpu.SemaphoreType.DMA for _ in range(NBUF)]
    ),
)


def _sigmoid(x):
    return 1.0 / (1.0 + jnp.exp(-x))


def _pack_pairs(h):
    u = lax.bitcast_convert_type(h, jnp.uint32)
    packed = (u[:, :PACK] >> 16) | (u[:, PACK:] & _HIMASK)
    return lax.bitcast_convert_type(packed, jnp.float32)


def _stage1_body(alo, ahi, x, wr_lo, wr_hi, wx, b,
                 out_lo, out_hi, out_plo, out_phi):
    acc = jnp.dot(alo[...], wr_lo[...], preferred_element_type=jnp.float32)
    acc += jnp.dot(ahi[...], wr_hi[...], preferred_element_type=jnp.float32)
    acc += jnp.dot(x[...], wx[...], preferred_element_type=jnp.float32)
    ht = _sigmoid(acc + b[...])
    hlo = ht[:, :HALF]
    hhi = ht[:, HALF:]
    out_lo[...] = hlo
    out_hi[...] = hhi
    out_plo[...] = _pack_pairs(hlo)
    out_phi[...] = _pack_pairs(hhi)


_stage1 = pl.pallas_call(
    _stage1_body,
    grid=(N // BLK,),
    in_specs=[
        pl.BlockSpec((BLK, HALF), lambda i: (i, 0)),
        pl.BlockSpec((BLK, HALF), lambda i: (i, 0)),
        pl.BlockSpec((BLK, D), lambda i: (i, 0)),
        pl.BlockSpec((HALF, D), lambda i: (0, 0)),
        pl.BlockSpec((HALF, D), lambda i: (0, 0)),
        pl.BlockSpec((D, D), lambda i: (0, 0)),
        pl.BlockSpec((1, D), lambda i: (0, 0)),
    ],
    out_specs=[
        pl.BlockSpec((BLK, HALF), lambda i: (i, 0)),
        pl.BlockSpec((BLK, HALF), lambda i: (i, 0)),
        pl.BlockSpec((BLK, PACK), lambda i: (i, 0)),
        pl.BlockSpec((BLK, PACK), lambda i: (i, 0)),
    ],
    out_shape=[
        jax.ShapeDtypeStruct((N, HALF), jnp.float32),
        jax.ShapeDtypeStruct((N, HALF), jnp.float32),
        jax.ShapeDtypeStruct((N, PACK), jnp.float32),
        jax.ShapeDtypeStruct((N, PACK), jnp.float32),
    ],
)


def _stage2_body(alo, ahi, hlo, hhi, wr_lo, wr_hi, wx_lo, wx_hi, b, out):
    acc = jnp.dot(alo[...], wr_lo[...], preferred_element_type=jnp.float32)
    acc += jnp.dot(ahi[...], wr_hi[...], preferred_element_type=jnp.float32)
    acc += jnp.dot(hlo[...], wx_lo[...], preferred_element_type=jnp.float32)
    acc += jnp.dot(hhi[...], wx_hi[...], preferred_element_type=jnp.float32)
    out[...] = _sigmoid(acc + b[...])


_stage2 = pl.pallas_call(
    _stage2_body,
    grid=(N // BLK,),
    in_specs=[
        pl.BlockSpec((BLK, HALF), lambda i: (i, 0)),
        pl.BlockSpec((BLK, HALF), lambda i: (i, 0)),
        pl.BlockSpec((BLK, HALF), lambda i: (i, 0)),
        pl.BlockSpec((BLK, HALF), lambda i: (i, 0)),
        pl.BlockSpec((HALF, D), lambda i: (0, 0)),
        pl.BlockSpec((HALF, D), lambda i: (0, 0)),
        pl.BlockSpec((HALF, D), lambda i: (0, 0)),
        pl.BlockSpec((HALF, D), lambda i: (0, 0)),
        pl.BlockSpec((1, D), lambda i: (0, 0)),
    ],
    out_specs=pl.BlockSpec((BLK, D), lambda i: (i, 0)),
    out_shape=jax.ShapeDtypeStruct((N, D), jnp.float32),
)


@jax.jit
def kernel(X, edge_index, edge_weight,
           W_hx_rel, b_hx_rel, W_hx_root,
           W_hh_rel, b_hh_rel, W_hh_root,
           W_y_rel, b_y_rel, W_y_root):
    src = edge_index[0].astype(jnp.int32)
    dst = edge_index[1].astype(jnp.int32)
    w = edge_weight.astype(jnp.float32)

    pad = E_PAD - E
    src = jnp.concatenate([src, jnp.zeros((pad,), jnp.int32)])
    dst = jnp.concatenate([dst, jnp.zeros((pad,), jnp.int32)])
    w = jnp.concatenate([w, jnp.zeros((pad,), jnp.float32)])

    xp2 = jnp.concatenate([_pack_pairs(X[:, :HALF]),
                           _pack_pairs(X[:, HALF:])], axis=0)

    agg2 = _segsum(xp2, src, dst, w)
    agg_lo = agg2[:N]
    agg_hi = agg2[ACCR:ACCR + N]

    wr = W_hx_rel.T
    b1 = (b_hx_rel + b_hh_rel).reshape(1, D)
    ht_lo, ht_hi, ht_plo, ht_phi = _stage1(
        agg_lo, agg_hi, X, wr[:HALF], wr[HALF:], W_hx_root.T, b1)

    htp2 = jnp.concatenate([ht_plo, ht_phi], axis=0)
    ah2 = _segsum(htp2, src, dst, w)
    ah_lo = ah2[:N]
    ah_hi = ah2[ACCR:ACCR + N]

    wyr = W_y_rel.T
    wyx = W_y_root.T
    yt = _stage2(ah_lo, ah_hi, ht_lo, ht_hi,
                 wyr[:HALF], wyr[HALF:], wyx[:HALF], wyx[HALF:],
                 b_y_rel.reshape(1, D))
    return yt
